# baseline (speedup 1.0000x reference)
import functools

import jax
import jax.numpy as jnp
from jax import lax
from jax.experimental import pallas as pl
from jax.experimental.pallas import tpu as pltpu
from jax.experimental.pallas import tpu_sc as plsc

ROWS = 128
COLS = 100000
NC, NS = 2, 16
NW = NC * NS
RPW = ROWS // NW
CH = 20000
NCHUNK = COLS // CH
LANES = 16
UNROLL = 10
NACC = 10
NEG = -3.0e38


def _body(inp_hbm, tgt_hbm, out_hbm, bufa, bufb, tgt_v, out_v, sema, semb):
    wid = lax.axis_index("c") * NS + lax.axis_index("s")
    pltpu.sync_copy(tgt_hbm, tgt_v)

    bufs = (bufa, bufb)
    sems = (sema, semb)

    def slot_src(s):
        i, c = divmod(s, NCHUNK)
        row = wid * RPW + i
        off = row * COLS + c * CH
        return inp_hbm.at[pl.ds(off, CH)]

    def start(s):
        pltpu.make_async_copy(slot_src(s), bufs[s % 2], sems[s % 2]).start()

    def wait(s):
        pltpu.make_async_copy(slot_src(s), bufs[s % 2], sems[s % 2]).wait()

    start(0)

    lane = lax.iota(jnp.int32, LANES)
    loss_acc = jnp.zeros((LANES,), jnp.float32)

    def _ins(a1, a2, a3, v):
        m = jnp.minimum(a1, v)
        a1 = jnp.maximum(a1, v)
        m2 = jnp.minimum(a2, m)
        a2 = jnp.maximum(a2, m)
        a3 = jnp.maximum(a3, m2)
        return a1, a2, a3

    for i in range(RPW):
        row = wid * RPW + i
        y_vec = plsc.load_gather(tgt_v, [jnp.broadcast_to(row, (LANES,))])
        neg = jnp.full((LANES,), NEG, jnp.float32)
        accs = [neg] * (3 * NACC)
        xy = jnp.zeros((LANES,), jnp.float32)

        for c in range(NCHUNK):
            s = i * NCHUNK + c
            if s + 1 < RPW * NCHUNK:
                start(s + 1)
            wait(s)
            buf = bufs[s % 2]

            def body(j, carry):
                acc = list(carry)
                for k in range(UNROLL):
                    v = buf[pl.ds(j * (LANES * UNROLL) + k * LANES, LANES)]
                    a = k % NACC
                    acc[3*a], acc[3*a+1], acc[3*a+2] = _ins(
                        acc[3*a], acc[3*a+1], acc[3*a+2], v)
                return tuple(acc)

            accs = list(lax.fori_loop(
                0, CH // (LANES * UNROLL), body, tuple(accs)))

            c0 = c * CH
            in_rng = (y_vec >= c0) & (y_vec < c0 + CH)
            off = jnp.minimum(jnp.maximum(y_vec - c0, 0), CH - 1)
            g = plsc.load_gather(buf, [off])
            xy = jnp.where(in_rng, g, xy)

        t1, t2, t3 = accs[0], accs[1], accs[2]
        for a in range(1, NACC):
            t1, t2, t3 = _ins(t1, t2, t3, accs[3*a])
            t1, t2, t3 = _ins(t1, t2, t3, accs[3*a+1])
            t1, t2, t3 = _ins(t1, t2, t3, accs[3*a+2])

        m1 = jnp.max(t1)
        m1s = jnp.broadcast_to(m1, (LANES,))
        c1 = (plsc.all_reduce_population_count(t1 == m1s)
              + plsc.all_reduce_population_count(t2 == m1s)
              + plsc.all_reduce_population_count(t3 == m1s))
        u1 = jnp.where(t1 < m1s, t1, NEG)
        u2 = jnp.where(t2 < m1s, t2, NEG)
        u3 = jnp.where(t3 < m1s, t3, NEG)
        n1 = jnp.maximum(jnp.maximum(jnp.max(u1), jnp.max(u2)), jnp.max(u3))
        n1s = jnp.broadcast_to(n1, (LANES,))
        c2 = (plsc.all_reduce_population_count(t1 == n1s)
              + plsc.all_reduce_population_count(t2 == n1s)
              + plsc.all_reduce_population_count(t3 == n1s))
        w1 = jnp.where(u1 < n1s, u1, NEG)
        w2 = jnp.where(u2 < n1s, u2, NEG)
        w3 = jnp.where(u3 < n1s, u3, NEG)
        n2 = jnp.maximum(jnp.maximum(jnp.max(w1), jnp.max(w2)), jnp.max(w3))
        n2s = jnp.broadcast_to(n2, (LANES,))

        m2s = jnp.where(c1 >= 2, m1s, n1s)
        m3s = jnp.where(c1 >= 3, m1s,
                        jnp.where((c1 == 2) | (c2 >= 2), n1s, n2s))

        ind = xy == m1s
        num = -(xy - jnp.where(ind, m2s, m1s))
        den = m1s - m3s + jnp.float32(1e-12)
        loss_acc = loss_acc + num / den

    out_v[...] = jnp.where(lane == 0, loss_acc, jnp.float32(0.0))
    pltpu.sync_copy(out_v, out_hbm.at[pl.ds(wid * LANES, LANES)])


@jax.jit
def _dlr_loss(inp_flat, target):
    out = pl.kernel(
        _body,
        out_type=jax.ShapeDtypeStruct((NW * LANES,), jnp.float32),
        mesh=plsc.VectorSubcoreMesh(
            core_axis_name="c", subcore_axis_name="s",
            num_cores=NC, num_subcores=NS),
        scratch_types=[
            pltpu.VMEM((CH,), jnp.float32),
            pltpu.VMEM((CH,), jnp.float32),
            pltpu.VMEM((ROWS,), jnp.int32),
            pltpu.VMEM((LANES,), jnp.float32),
            pltpu.SemaphoreType.DMA,
            pltpu.SemaphoreType.DMA,
        ],
        compiler_params=pltpu.CompilerParams(needs_layout_passes=False),
    )(inp_flat, target)
    return jnp.sum(out) / jnp.float32(ROWS)


def kernel(input, target):
    return _dlr_loss(input.reshape(-1), target)


# speedup vs baseline: 1.6780x; 1.6780x over previous
import jax
import jax.numpy as jnp
from jax import lax
from jax.experimental import pallas as pl
from jax.experimental.pallas import tpu as pltpu
from jax.experimental.pallas import tpu_sc as plsc

ROWS = 128
COLS = 100000
PAD_COLS = 100096
HALF_W = 50048
CH = 2944
NCH = HALF_W // CH
LANES = 16
GR = 8
NEG = -3.0e38


def _ins(a1, a2, a3, v):
    m = jnp.minimum(a1, v)
    a1 = jnp.maximum(a1, v)
    m2 = jnp.minimum(a2, m)
    a2 = jnp.maximum(a2, m)
    a3 = jnp.maximum(a3, m2)
    return a1, a2, a3


def _body(inp_hbm, tgt_hbm, out_hbm, xch_hbm,
          bufa, bufb, tgt_v, res_v, rb_v, partner_v, out_v,
          sema, semb):
    c = lax.axis_index("c")
    s = lax.axis_index("s")
    wid = c * 16 + s
    g = c * 8 + s // 2
    h = s % 2
    r0 = g * GR
    base_col = h * HALF_W

    pltpu.sync_copy(tgt_hbm, tgt_v)

    bufs = (bufa, bufb)
    sems = (sema, semb)

    def src(k):
        return inp_hbm.at[pl.ds(r0, GR), pl.ds(base_col + k * CH, CH)]

    def start(k):
        pltpu.make_async_copy(src(k), bufs[k % 2], sems[k % 2]).start()

    def wait(k):
        pltpu.make_async_copy(src(k), bufs[k % 2], sems[k % 2]).wait()

    lane = lax.iota(jnp.int32, LANES)
    lane_row = jnp.minimum(lane, GR - 1)
    tgt_grp = plsc.load_gather(tgt_v, [jnp.minimum(r0 + lane, ROWS - 1)])
    hi_col = jnp.where(h == 0, jnp.int32(1 << 30), jnp.int32(COLS))

    neg = jnp.full((LANES,), NEG, jnp.float32)
    accs = [neg] * (3 * GR)
    xy_grp = jnp.zeros((LANES,), jnp.float32)

    start(0)
    for k in range(NCH):
        if k + 1 < NCH:
            start(k + 1)
        wait(k)
        buf = bufs[k % 2]
        coff = base_col + k * CH
        mask_last = (k == NCH - 1)

        def body(j, carry, buf=buf, coff=coff, mask_last=mask_last):
            acc = list(carry)
            if mask_last:
                colv = coff + j * LANES + lane
                okhi = colv < hi_col
            for r in range(GR):
                v = buf[r, pl.ds(j * LANES, LANES)]
                if mask_last:
                    v = jnp.where(okhi, v, NEG)
                acc[3*r], acc[3*r+1], acc[3*r+2] = _ins(
                    acc[3*r], acc[3*r+1], acc[3*r+2], v)
            return tuple(acc)

        accs = list(lax.fori_loop(0, CH // LANES, body, tuple(accs)))

        in_ch = (tgt_grp >= coff) & (tgt_grp < coff + CH) & (lane < GR)
        colg = jnp.minimum(jnp.maximum(tgt_grp - coff, 0), CH - 1)
        gath = plsc.load_gather(buf, [lane_row, colg])
        xy_grp = jnp.where(in_ch, gath, xy_grp)

    row_res = []
    for r in range(GR):
        t1, t2, t3 = accs[3*r], accs[3*r+1], accs[3*r+2]
        m1 = jnp.max(t1)
        m1s = jnp.broadcast_to(m1, (LANES,))
        c1 = (plsc.all_reduce_population_count(t1 == m1s)
              + plsc.all_reduce_population_count(t2 == m1s)
              + plsc.all_reduce_population_count(t3 == m1s))
        u1 = jnp.where(t1 < m1s, t1, neg)
        u2 = jnp.where(t2 < m1s, t2, neg)
        u3 = jnp.where(t3 < m1s, t3, neg)
        n1 = jnp.maximum(jnp.maximum(jnp.max(u1), jnp.max(u2)), jnp.max(u3))
        n1s = jnp.broadcast_to(n1, (LANES,))
        c2 = (plsc.all_reduce_population_count(t1 == n1s)
              + plsc.all_reduce_population_count(t2 == n1s)
              + plsc.all_reduce_population_count(t3 == n1s))
        w1 = jnp.where(u1 < n1s, u1, neg)
        w2 = jnp.where(u2 < n1s, u2, neg)
        w3 = jnp.where(u3 < n1s, u3, neg)
        n2 = jnp.maximum(jnp.maximum(jnp.max(w1), jnp.max(w2)), jnp.max(w3))
        n2s = jnp.broadcast_to(n2, (LANES,))
        m2s = jnp.where(c1 >= 2, m1s, n1s)
        m3s = jnp.where(c1 >= 3, m1s,
                        jnp.where((c1 == 2) | (c2 >= 2), n1s, n2s))
        xys = jnp.take(xy_grp, jnp.full((LANES,), r, jnp.int32))
        row_res.append((m1s, m2s, m3s, xys))

    v0 = jnp.zeros((LANES,), jnp.float32)
    v1 = jnp.zeros((LANES,), jnp.float32)
    for r in range(4):
        for comp in range(4):
            v0 = jnp.where(lane == 4*r + comp, row_res[r][comp], v0)
            v1 = jnp.where(lane == 4*r + comp, row_res[4 + r][comp], v1)
    res_v[pl.ds(0, LANES)] = v0
    res_v[pl.ds(LANES, LANES)] = v1

    def _pub_cond(carry):
        it, ok = carry
        return (~ok) & (it < 8)

    def _pub_body(carry):
        it, ok = carry
        pltpu.sync_copy(res_v, xch_hbm.at[pl.ds(wid * 2 * LANES, 2 * LANES)])
        pltpu.sync_copy(xch_hbm.at[pl.ds(wid * 2 * LANES, 2 * LANES)], rb_v)
        ra = rb_v[pl.ds(0, LANES)]
        rb = rb_v[pl.ds(LANES, LANES)]
        ok = jnp.all(ra == v0) & jnp.all(rb == v1)
        return it + 1, ok

    lax.while_loop(_pub_cond, _pub_body, (jnp.int32(0), jnp.bool_(False)))
    plsc.subcore_barrier()
    pltpu.sync_copy(
        xch_hbm.at[pl.ds((wid ^ 1) * 2 * LANES, 2 * LANES)], partner_v)
    pa = partner_v[pl.ds(0, LANES)]
    pb = partner_v[pl.ds(LANES, LANES)]

    loss_acc = jnp.zeros((LANES,), jnp.float32)
    for r in range(GR):
        a1, a2, a3, axy = row_res[r]
        src_half = pa if r < 4 else pb
        q = 4 * (r % 4)
        b1 = jnp.take(src_half, jnp.full((LANES,), q, jnp.int32))
        b2 = jnp.take(src_half, jnp.full((LANES,), q + 1, jnp.int32))
        b3 = jnp.take(src_half, jnp.full((LANES,), q + 2, jnp.int32))
        bxy = jnp.take(src_half, jnp.full((LANES,), q + 3, jnp.int32))
        t1, t2, t3 = _ins(a1, a2, a3, b1)
        t1, t2, t3 = _ins(t1, t2, t3, b2)
        t1, t2, t3 = _ins(t1, t2, t3, b3)
        y_r = plsc.load_gather(tgt_v, [jnp.full((LANES,), 0, jnp.int32) + r0 + r])
        in_mine = (y_r >= base_col) & (y_r < base_col + HALF_W)
        xy = jnp.where(in_mine, axy, bxy)
        ind = xy == t1
        num = -(xy - jnp.where(ind, t2, t1))
        den = t1 - t3 + jnp.float32(1e-12)
        loss_acc = loss_acc + jnp.where((lane == 0) & (h == 0),
                                        num / den, jnp.float32(0.0))
    out_v[...] = loss_acc
    pltpu.sync_copy(out_v, out_hbm.at[pl.ds(wid * LANES, LANES)])


@jax.jit
def _dlr_loss(inp, target):
    out, _ = pl.kernel(
        _body,
        out_type=(jax.ShapeDtypeStruct((32 * LANES,), jnp.float32),
                  jax.ShapeDtypeStruct((32 * 2 * LANES,), jnp.float32)),
        mesh=plsc.VectorSubcoreMesh(
            core_axis_name="c", subcore_axis_name="s",
            num_cores=2, num_subcores=16),
        scratch_types=[
            pltpu.VMEM((GR, CH), jnp.float32),
            pltpu.VMEM((GR, CH), jnp.float32),
            pltpu.VMEM((ROWS,), jnp.int32),
            pltpu.VMEM((2 * LANES,), jnp.float32),
            pltpu.VMEM((2 * LANES,), jnp.float32),
            pltpu.VMEM((2 * LANES,), jnp.float32),
            pltpu.VMEM((LANES,), jnp.float32),
            pltpu.SemaphoreType.DMA,
            pltpu.SemaphoreType.DMA,
        ],
        compiler_params=pltpu.CompilerParams(needs_layout_passes=False),
    )(inp, target)
    return jnp.sum(out) / jnp.float32(ROWS)


def kernel(input, target):
    return _dlr_loss(input, target)
